# baseline (speedup 1.0000x reference)
import jax
import jax.numpy as jnp
from jax.experimental import pallas as pl

_BLOCK_ROWS = 8192


def _assign_body(v_ref, ct_ref, out_ref):
    v = v_ref[...]
    sq = jnp.sum(v * v, axis=1, keepdims=True)
    vn = v / jnp.maximum(jnp.sqrt(sq), 1e-12)
    scores = jnp.dot(vn, ct_ref[...], preferred_element_type=jnp.float32)
    out_ref[...] = jnp.argmax(scores, axis=1).astype(jnp.int32)


@jax.jit
def _assign(vectors, centroids_t):
    n, d = vectors.shape
    k = centroids_t.shape[1]
    b = _BLOCK_ROWS
    grid = n // b
    return pl.pallas_call(
        _assign_body,
        grid=(grid,),
        in_specs=[
            pl.BlockSpec((b, d), lambda i: (i, 0)),
            pl.BlockSpec((d, k), lambda i: (0, 0)),
        ],
        out_specs=pl.BlockSpec((b,), lambda i: (i,)),
        out_shape=jax.ShapeDtypeStruct((n,), jnp.int32),
    )(vectors, centroids_t)


def kernel(vectors, centroids):
    return _assign(vectors, centroids.T)


# speedup vs baseline: 2.2208x; 2.2208x over previous
import jax
import jax.numpy as jnp
from jax.experimental import pallas as pl

_BLOCK_ROWS = 8192


def _assign_body(v_ref, c_ref, out_ref):
    scores = jax.lax.dot_general(
        c_ref[...], v_ref[...],
        dimension_numbers=(((1,), (1,)), ((), ())),
        preferred_element_type=jnp.float32,
    )
    out_ref[...] = jnp.argmax(scores, axis=0).astype(jnp.int32)


@jax.jit
def _assign(vectors, centroids):
    n, d = vectors.shape
    k = centroids.shape[0]
    b = _BLOCK_ROWS
    grid = n // b
    return pl.pallas_call(
        _assign_body,
        grid=(grid,),
        in_specs=[
            pl.BlockSpec((b, d), lambda i: (i, 0)),
            pl.BlockSpec((k, d), lambda i: (0, 0)),
        ],
        out_specs=pl.BlockSpec((b,), lambda i: (i,)),
        out_shape=jax.ShapeDtypeStruct((n,), jnp.int32),
    )(vectors, centroids)


def kernel(vectors, centroids):
    return _assign(vectors, centroids)
